# baseline (speedup 1.0000x reference)
import jax
import jax.numpy as jnp
from jax.experimental import pallas as pl
from jax.experimental.pallas import tpu as pltpu

NUM_EXPERTS = 16
TOP_K = 2
ROW_TILE = 1024


def _router_body(x_ref, w_ref, b_ref, idx_ref, wgt_ref, logit_ref):
    i = pl.program_id(0)
    y = jax.lax.dot_general(
        w_ref[...], x_ref[...],
        (((1,), (1,)), ((), ())),
        preferred_element_type=jnp.float32,
    )
    y = y + b_ref[...]
    bits = (y > 0).astype(jnp.int32)
    s0 = jnp.sum(bits[:32, :], axis=0, keepdims=True)
    s1 = jnp.sum(bits[32:, :], axis=0, keepdims=True)
    r0 = jnp.bitwise_and(s0, NUM_EXPERTS - 1)
    r1 = jnp.bitwise_and(s1, NUM_EXPERTS - 1)
    i1 = jnp.where(r1 == r0, jnp.bitwise_and(r0 + 1, NUM_EXPERTS - 1), r1)
    idx_t = jnp.concatenate([r0, i1], axis=0)
    idx_ref[pl.ds(i * ROW_TILE, ROW_TILE), :] = idx_t.T

    @pl.when(i == 0)
    def _fill_constants():
        wgt_ref[...] = jnp.full(wgt_ref.shape, 1.0 / TOP_K, dtype=jnp.float32)
        logit_ref[...] = jnp.zeros(logit_ref.shape, dtype=jnp.float32)


def kernel(hidden_states, W, b):
    B, S, H = hidden_states.shape
    T = B * S
    x = hidden_states.reshape(T, H)
    grid = (T // ROW_TILE,)
    idx, wgt, logits = pl.pallas_call(
        _router_body,
        grid=grid,
        in_specs=[
            pl.BlockSpec((ROW_TILE, H), lambda i: (i, 0)),
            pl.BlockSpec((TOP_K * 32, H), lambda i: (0, 0)),
            pl.BlockSpec((TOP_K * 32, 1), lambda i: (0, 0)),
        ],
        out_specs=[
            pl.BlockSpec((T, TOP_K), lambda i: (0, 0)),
            pl.BlockSpec((T * TOP_K // 128, 128), lambda i: (0, 0)),
            pl.BlockSpec((T * NUM_EXPERTS // 128, 128), lambda i: (0, 0)),
        ],
        out_shape=[
            jax.ShapeDtypeStruct((T, TOP_K), jnp.int32),
            jax.ShapeDtypeStruct((T * TOP_K // 128, 128), jnp.float32),
            jax.ShapeDtypeStruct((T * NUM_EXPERTS // 128, 128), jnp.float32),
        ],
        compiler_params=pltpu.CompilerParams(
            dimension_semantics=("arbitrary",),
        ),
    )(x, W, b.reshape(TOP_K * 32, 1))
    expert_indices = idx.astype(jnp.int64)
    expert_weights = wgt.reshape(T, TOP_K)
    router_logits = logits.reshape(T, NUM_EXPERTS)
    return (expert_weights, expert_indices, router_logits)


# speedup vs baseline: 1.1232x; 1.1232x over previous
import jax
import jax.numpy as jnp
from jax.experimental import pallas as pl
from jax.experimental.pallas import tpu as pltpu

NUM_EXPERTS = 16
TOP_K = 2
ROW_TILE = 1024


def _router_body(x_ref, w_ref, b_ref, idx_ref, wgt_ref, logit_ref):
    y = jax.lax.dot_general(
        w_ref[...], x_ref[...],
        (((1,), (1,)), ((), ())),
        preferred_element_type=jnp.float32,
    )
    y = y + b_ref[...]
    bits = (y > 0).astype(jnp.int32)
    s0 = jnp.sum(bits[:32, :], axis=0, keepdims=True)
    s1 = jnp.sum(bits[32:, :], axis=0, keepdims=True)
    r0 = jnp.bitwise_and(s0, NUM_EXPERTS - 1)
    r1 = jnp.bitwise_and(s1, NUM_EXPERTS - 1)
    i1 = jnp.where(r1 == r0, jnp.bitwise_and(r0 + 1, NUM_EXPERTS - 1), r1)
    idx_ref[...] = jnp.concatenate([r0, i1], axis=0)

    @pl.when(pl.program_id(0) == 0)
    def _fill_constants():
        wgt_ref[...] = jnp.full(wgt_ref.shape, 1.0 / TOP_K, dtype=jnp.float32)
        logit_ref[...] = jnp.zeros(logit_ref.shape, dtype=jnp.float32)


def kernel(hidden_states, W, b):
    B, S, H = hidden_states.shape
    T = B * S
    x = hidden_states.reshape(T, H)
    grid = (T // ROW_TILE,)
    idx_t, wgt, logits = pl.pallas_call(
        _router_body,
        grid=grid,
        in_specs=[
            pl.BlockSpec((ROW_TILE, H), lambda i: (i, 0)),
            pl.BlockSpec((TOP_K * 32, H), lambda i: (0, 0)),
            pl.BlockSpec((TOP_K * 32, 1), lambda i: (0, 0)),
        ],
        out_specs=[
            pl.BlockSpec((TOP_K, ROW_TILE), lambda i: (0, i)),
            pl.BlockSpec((T * TOP_K // 128, 128), lambda i: (0, 0)),
            pl.BlockSpec((T * NUM_EXPERTS // 128, 128), lambda i: (0, 0)),
        ],
        out_shape=[
            jax.ShapeDtypeStruct((TOP_K, T), jnp.int32),
            jax.ShapeDtypeStruct((T * TOP_K // 128, 128), jnp.float32),
            jax.ShapeDtypeStruct((T * NUM_EXPERTS // 128, 128), jnp.float32),
        ],
        compiler_params=pltpu.CompilerParams(
            dimension_semantics=("arbitrary",),
        ),
    )(x, W, b.reshape(TOP_K * 32, 1))
    expert_indices = idx_t.T.astype(jnp.int64)
    expert_weights = wgt.reshape(T, TOP_K)
    router_logits = logits.reshape(T, NUM_EXPERTS)
    return (expert_weights, expert_indices, router_logits)


# speedup vs baseline: 1.5091x; 1.3436x over previous
import jax
import jax.numpy as jnp
from jax.experimental import pallas as pl
from jax.experimental.pallas import tpu as pltpu

NUM_EXPERTS = 16
TOP_K = 2
ROW_TILE = 1024


def _router_body(x_ref, w_ref, b_ref, idx_ref):
    y = jax.lax.dot_general(
        w_ref[...], x_ref[...],
        (((1,), (1,)), ((), ())),
        preferred_element_type=jnp.float32,
    )
    y = y + b_ref[...]
    bits = (y > 0).astype(jnp.int32)
    s0 = jnp.sum(bits[:32, :], axis=0, keepdims=True)
    s1 = jnp.sum(bits[32:, :], axis=0, keepdims=True)
    r0 = jnp.bitwise_and(s0, NUM_EXPERTS - 1)
    r1 = jnp.bitwise_and(s1, NUM_EXPERTS - 1)
    i1 = jnp.where(r1 == r0, jnp.bitwise_and(r0 + 1, NUM_EXPERTS - 1), r1)
    idx_ref[...] = jnp.concatenate([r0, i1], axis=0)


def kernel(hidden_states, W, b):
    B, S, H = hidden_states.shape
    T = B * S
    x = hidden_states.reshape(T, H)
    grid = (T // ROW_TILE,)
    idx_t = pl.pallas_call(
        _router_body,
        grid=grid,
        in_specs=[
            pl.BlockSpec((ROW_TILE, H), lambda i: (i, 0)),
            pl.BlockSpec((TOP_K * 32, H), lambda i: (0, 0)),
            pl.BlockSpec((TOP_K * 32, 1), lambda i: (0, 0)),
        ],
        out_specs=pl.BlockSpec((TOP_K, ROW_TILE), lambda i: (0, i)),
        out_shape=jax.ShapeDtypeStruct((TOP_K, T), jnp.int32),
        compiler_params=pltpu.CompilerParams(
            dimension_semantics=("parallel",),
        ),
    )(x, W, b.reshape(TOP_K * 32, 1))
    expert_indices = idx_t.T.astype(jnp.int64)
    expert_weights = jnp.full((T, TOP_K), 1.0 / TOP_K, dtype=jnp.float32)
    router_logits = jnp.zeros((T, NUM_EXPERTS), dtype=jnp.float32)
    return (expert_weights, expert_indices, router_logits)
